# baseline (speedup 1.0000x reference)
import jax
import jax.numpy as jnp
from jax import lax
from jax.experimental import pallas as pl
from jax.experimental.pallas import tpu as pltpu
from jax.experimental.pallas import tpu_sc as plsc
from jax._src.pallas import mpmd as _mpmd

DEPTH = 1000
B_CONST = 1024
N_WORDS = 50 * DEPTH * B_CONST
FILL_CHUNK = 640000
FILL_NBUF = 8
L = 16
ROWS_PER_W = 1600


def _fill_body(off_ref, out_ref, *scratch):
    bufs = scratch[:FILL_NBUF]
    sems = scratch[FILL_NBUF:]
    off = off_ref[0, 0]
    for k in range(FILL_NBUF):
        bufs[k][...] = jnp.full((FILL_CHUNK,), off, jnp.float32)

    n_outer = N_WORDS // FILL_CHUNK // FILL_NBUF

    def outer(i, carry):
        for k in range(FILL_NBUF):
            c = i * FILL_NBUF + k

            @pl.when(i > 0)
            def _wait():
                pltpu.make_async_copy(
                    bufs[k], out_ref.at[pl.ds(c * FILL_CHUNK, FILL_CHUNK)], sems[k]
                ).wait()

            pltpu.make_async_copy(
                bufs[k], out_ref.at[pl.ds(c * FILL_CHUNK, FILL_CHUNK)], sems[k]
            ).start()
        return carry

    lax.fori_loop(0, n_outer, outer, 0)
    for k in range(FILL_NBUF):
        pltpu.make_async_copy(
            bufs[k], out_ref.at[pl.ds(0, FILL_CHUNK)], sems[k]
        ).wait()


def _scatter_body(filled_hbm, xf_hbm, on_hbm, out_hbm, xflat_v, onbuf_v, idx_v):
    del filled_hbm
    info = plsc.get_sparse_core_info()
    nc = info.num_cores
    s = 50
    wid = lax.axis_index("s") * nc + lax.axis_index("c")
    q0 = wid * ROWS_PER_W
    i0 = wid * 32

    pltpu.sync_copy(xf_hbm.at[pl.ds(q0, ROWS_PER_W)], xflat_v)
    pltpu.sync_copy(on_hbm, onbuf_v)
    lanes = lax.iota(jnp.int32, L)

    def build(t, _):
        v = xflat_v[pl.ds(t * L, L)]
        q = t * L + lanes
        iv = i0 + q // s
        jv = q % s
        idx16 = (
            jv * (DEPTH * B_CONST)
            + ((v >> 3) << 13)
            + ((iv >> 7) << 10)
            + ((v & 7) << 7)
            + (iv & 127)
        )
        p = t * L
        idx_v[p >> 7, pl.ds(p & 127, L)] = idx16

        @pl.when(t >= (ROWS_PER_W // L) - 4)
        def _pad():
            p2 = p + 64
            idx_v[p2 >> 7, pl.ds(p2 & 127, L)] = idx16

        return 0

    lax.fori_loop(0, ROWS_PER_W // L, build, 0)

    def fire(sem):
        for r in range(13):
            pltpu.make_async_copy(
                onbuf_v.at[r], out_hbm.at[idx_v.at[r]], sem
            ).start()
        for r in range(13):
            pltpu.make_async_copy(
                onbuf_v.at[r], out_hbm.at[idx_v.at[r]], sem
            ).wait()

    pl.run_scoped(fire, pltpu.SemaphoreType.DMA)


def kernel(x, on_value, off_value):
    B, S = x.shape
    offv = jnp.asarray(off_value, jnp.float32).reshape(1, 1)
    filled = pl.pallas_call(
        _fill_body,
        in_specs=[pl.BlockSpec(memory_space=pltpu.SMEM)],
        out_specs=pl.BlockSpec(memory_space=pl.ANY),
        out_shape=jax.ShapeDtypeStruct((N_WORDS,), jnp.float32),
        scratch_shapes=(
            [pltpu.VMEM((FILL_CHUNK,), jnp.float32)] * FILL_NBUF
            + [pltpu.SemaphoreType.DMA] * FILL_NBUF
        ),
    )(offv)

    xf = x.reshape(B * S)
    on13x128 = jnp.full((13, 128), on_value, jnp.float32)
    mesh = plsc.VectorSubcoreMesh(core_axis_name="c", subcore_axis_name="s")
    f = _mpmd._mpmd_map(
        [(mesh, _scatter_body)],
        jax.ShapeDtypeStruct((N_WORDS,), jnp.float32),
        input_output_aliases={0: 0},
        compiler_params=pltpu.CompilerParams(
            use_tc_tiling_on_sc=False, needs_layout_passes=False
        ),
        scratch_types=[
            pltpu.VMEM((ROWS_PER_W,), jnp.int32),
            pltpu.VMEM((13, 128), jnp.float32),
            pltpu.VMEM((13, 128), jnp.int32),
        ],
    )
    out = f(filled, xf, on13x128)
    out5 = out.reshape(S, DEPTH // 8, B // 128, 8, 128)
    return out5.transpose(2, 4, 0, 1, 3).reshape(B, S, DEPTH)


# speedup vs baseline: 1.1142x; 1.1142x over previous
import jax
import jax.numpy as jnp
from jax import lax
from jax.experimental import pallas as pl
from jax.experimental.pallas import tpu as pltpu
from jax.experimental.pallas import tpu_sc as plsc

DEPTH = 1000
B_CONST = 1024
DCHUNK = 40
CHUNK_WORDS = DCHUNK * B_CONST
NCHUNKS_PER_SLAB = DEPTH // DCHUNK
L = 16


def _sc_body(
    xt_hbm, on_hbm, off_hbm, out_hbm, xc0, xc1, buf0, buf1, on_v, off_v, s0, s1
):
    info = plsc.get_sparse_core_info()
    nc = info.num_cores
    nw = nc * info.num_subcores
    s, b = xt_hbm.shape
    n_chunks = s * NCHUNKS_PER_SLAB
    wid = lax.axis_index("s") * nc + lax.axis_index("c")
    xcols = (xc0, xc1)
    bufs = (buf0, buf1)
    sems = (s0, s1)
    n_m = pl.cdiv(n_chunks, nw)

    pltpu.sync_copy(on_hbm, on_v)
    pltpu.sync_copy(off_hbm, off_v)
    on_vec = on_v[...]
    off_vec = off_v[...]

    def fill(t, _):
        buf0[pl.ds(t * L, L)] = off_vec
        buf1[pl.ds(t * L, L)] = off_vec
        return 0

    lax.fori_loop(0, CHUNK_WORDS // L, fill, 0)

    lanes = lax.iota(jnp.int32, L)

    def run(t, _):
        for bb in range(2):
            m = 2 * t + bb
            c = wid + nw * m
            xcol = xcols[bb]
            buf = bufs[bb]
            sem = sems[bb]

            @pl.when(c < n_chunks)
            def _():
                @pl.when(t > 0)
                def _wait_restore():
                    pltpu.make_async_copy(
                        buf, out_hbm.at[pl.ds(c * CHUNK_WORDS, CHUNK_WORDS)], sem
                    ).wait()
                    c_prev = c - 2 * nw
                    dlo_prev = (c_prev % NCHUNKS_PER_SLAB) * DCHUNK

                    def restore(k, v_carry):
                        v = xcol[pl.ds(k * L, L)]
                        ld = v - dlo_prev
                        mask = (ld >= 0) & (ld < DCHUNK)
                        idx = (
                            ((ld >> 3) << 13)
                            + ((ld & 7) << 7)
                            + ((k >> 3) << 10)
                            + ((k & 7) << 4)
                            + lanes
                        )
                        plsc.store_scatter(buf, [idx], v_carry, mask=mask)
                        return v_carry

                    lax.fori_loop(0, b // L, restore, off_vec)

                j = c // NCHUNKS_PER_SLAB
                dlo = (c % NCHUNKS_PER_SLAB) * DCHUNK
                pltpu.sync_copy(xt_hbm.at[j], xcol)

                def poke_on(k, v_carry):
                    v = xcol[pl.ds(k * L, L)]
                    ld = v - dlo
                    mask = (ld >= 0) & (ld < DCHUNK)
                    idx = (
                        ((ld >> 3) << 13)
                        + ((ld & 7) << 7)
                        + ((k >> 3) << 10)
                        + ((k & 7) << 4)
                        + lanes
                    )
                    plsc.store_scatter(buf, [idx], v_carry, mask=mask)
                    return v_carry

                lax.fori_loop(0, b // L, poke_on, on_vec)
                pltpu.make_async_copy(
                    buf, out_hbm.at[pl.ds(c * CHUNK_WORDS, CHUNK_WORDS)], sem
                ).start()

        return 0

    lax.fori_loop(0, pl.cdiv(n_m, 2), run, 0)

    for bb in range(2):
        last_c = wid
        pltpu.make_async_copy(
            bufs[bb], out_hbm.at[pl.ds(last_c * 0, CHUNK_WORDS)], sems[bb]
        ).wait()


def kernel(x, on_value, off_value):
    B, S = x.shape
    xt = x.T
    on16 = jnp.full((L,), on_value, jnp.float32)
    off16 = jnp.full((L,), off_value, jnp.float32)
    mesh = plsc.VectorSubcoreMesh(core_axis_name="c", subcore_axis_name="s")
    f = pl.kernel(
        _sc_body,
        out_type=jax.ShapeDtypeStruct((S * DEPTH * B,), jnp.float32),
        mesh=mesh,
        compiler_params=pltpu.CompilerParams(
            use_tc_tiling_on_sc=False, needs_layout_passes=False
        ),
        scratch_types=[
            pltpu.VMEM((B_CONST,), jnp.int32),
            pltpu.VMEM((B_CONST,), jnp.int32),
            pltpu.VMEM((CHUNK_WORDS,), jnp.float32),
            pltpu.VMEM((CHUNK_WORDS,), jnp.float32),
            pltpu.VMEM((L,), jnp.float32),
            pltpu.VMEM((L,), jnp.float32),
            pltpu.SemaphoreType.DMA,
            pltpu.SemaphoreType.DMA,
        ],
    )
    out = f(xt, on16, off16)
    out5 = out.reshape(S, DEPTH // 8, B // 128, 8, 128)
    return out5.transpose(2, 4, 0, 1, 3).reshape(B, S, DEPTH)
